# Initial kernel scaffold; baseline (speedup 1.0000x reference)
import jax
import jax.numpy as jnp
from jax.experimental import pallas as pl


def kernel(x):
    raise NotImplementedError("write your pallas kernel here")


# speedup vs baseline: 6.3671x; 6.3671x over previous
import jax
import jax.numpy as jnp
from jax.experimental import pallas as pl

_HB = 8


def _flip_body(x_ref, o_ref):
    for i in range(_HB):
        o_ref[:, i, :] = x_ref[:, _HB - 1 - i, :]


def kernel(x):
    B, C, D, H, W = x.shape
    L = B * C * D
    xr = x.reshape(L, H, W)
    Lb = 256
    nH = H // _HB
    out = pl.pallas_call(
        _flip_body,
        grid=(L // Lb, nH),
        in_specs=[pl.BlockSpec((Lb, _HB, W), lambda l, h: (l, h, 0))],
        out_specs=pl.BlockSpec((Lb, _HB, W), lambda l, h: (l, nH - 1 - h, 0)),
        out_shape=jax.ShapeDtypeStruct((L, H, W), x.dtype),
    )(xr)
    return out.reshape(B, C, D, H, W)
